# initial kernel scaffold (unmeasured)
import jax
import jax.numpy as jnp
from jax import lax
from jax.experimental import pallas as pl
from jax.experimental.pallas import tpu as pltpu


def kernel(
    x,
):
    def body(*refs):
        pass

    out_shape = jax.ShapeDtypeStruct(..., jnp.float32)
    return pl.pallas_call(body, out_shape=out_shape)(...)


# baseline (device time: 2127478 ns/iter reference)
import jax
import jax.numpy as jnp
from jax import lax
from jax.experimental import pallas as pl
from jax.experimental.pallas import tpu as pltpu


def kernel(x):
    m_per, n = x.shape
    half = m_per // 2

    def body(x_ref, out_ref, local_sem, send_sems, recv_sems):
        my_x = lax.axis_index("x")
        my_y = lax.axis_index("y")

        barrier_sem = pltpu.get_barrier_semaphore()
        pl.semaphore_signal(barrier_sem, inc=1,
                            device_id=(1 - my_x, my_y),
                            device_id_type=pl.DeviceIdType.MESH)
        pl.semaphore_signal(barrier_sem, inc=1,
                            device_id=(my_x, 1 - my_y),
                            device_id_type=pl.DeviceIdType.MESH)
        pl.semaphore_wait(barrier_sem, 2)

        local_copy = pltpu.make_async_copy(
            x_ref.at[:, :],
            out_ref.at[pl.ds(my_x * m_per, m_per), :],
            local_sem,
        )
        local_copy.start()

        send_off = my_y * half
        own_out_off = my_x * m_per + my_y * half
        rdma1 = pltpu.make_async_remote_copy(
            src_ref=x_ref.at[pl.ds(send_off, half), :],
            dst_ref=out_ref.at[pl.ds(own_out_off, half), :],
            send_sem=send_sems.at[0],
            recv_sem=recv_sems.at[0],
            device_id=(1 - my_x, my_y),
            device_id_type=pl.DeviceIdType.MESH,
        )
        rdma1.start()
        rdma1.wait()

        fwd_off = (1 - my_x) * m_per + my_y * half
        rdma2 = pltpu.make_async_remote_copy(
            src_ref=out_ref.at[pl.ds(fwd_off, half), :],
            dst_ref=out_ref.at[pl.ds(fwd_off, half), :],
            send_sem=send_sems.at[1],
            recv_sem=recv_sems.at[1],
            device_id=(my_x, 1 - my_y),
            device_id_type=pl.DeviceIdType.MESH,
        )
        rdma2.start()
        rdma2.wait()

        local_copy.wait()

    return pl.pallas_call(
        body,
        out_shape=jax.ShapeDtypeStruct((2 * m_per, n), x.dtype),
        in_specs=[pl.BlockSpec(memory_space=pl.ANY)],
        out_specs=pl.BlockSpec(memory_space=pl.ANY),
        scratch_shapes=[
            pltpu.SemaphoreType.DMA,
            pltpu.SemaphoreType.DMA((2,)),
            pltpu.SemaphoreType.DMA((2,)),
        ],
        compiler_params=pltpu.CompilerParams(collective_id=0),
    )(x)


# device time: 476067 ns/iter; 4.4689x vs baseline; 4.4689x over previous
import jax
import jax.numpy as jnp
from jax import lax
from jax.experimental import pallas as pl
from jax.experimental.pallas import tpu as pltpu

NC = 16
LC = 32
NSLOT = 4
PRE = 4


def kernel(x):
    m_per, n = x.shape
    half = m_per // 2
    hc = half // NC
    lc = m_per // LC

    def body(x_ref, out_ref, vmem, in_sems, out_sems,
             send1_sems, recv1_sems, send2_sems, recv2_sems):
        my_x = lax.axis_index("x")
        my_y = lax.axis_index("y")

        barrier_sem = pltpu.get_barrier_semaphore()
        pl.semaphore_signal(barrier_sem, inc=1,
                            device_id=(1 - my_x, my_y),
                            device_id_type=pl.DeviceIdType.MESH)
        pl.semaphore_signal(barrier_sem, inc=1,
                            device_id=(my_x, 1 - my_y),
                            device_id_type=pl.DeviceIdType.MESH)
        pl.semaphore_wait(barrier_sem, 2)

        send_base = my_y * half
        own_base = my_x * m_per + my_y * half
        hop1 = []
        for k in range(NC):
            r = pltpu.make_async_remote_copy(
                src_ref=x_ref.at[pl.ds(send_base + k * hc, hc), :],
                dst_ref=out_ref.at[pl.ds(own_base + k * hc, hc), :],
                send_sem=send1_sems.at[k],
                recv_sem=recv1_sems.at[k],
                device_id=(1 - my_x, my_y),
                device_id_type=pl.DeviceIdType.MESH,
            )
            r.start()
            hop1.append(r)

        local = []

        def local_chunk(j):
            slot = j % NSLOT
            if j >= NSLOT:
                local[j - NSLOT].wait()
            cp_in = pltpu.make_async_copy(
                x_ref.at[pl.ds(j * lc, lc), :],
                vmem.at[slot], in_sems.at[slot])
            cp_in.start()
            cp_in.wait()
            cp_out = pltpu.make_async_copy(
                vmem.at[slot],
                out_ref.at[pl.ds(my_x * m_per + j * lc, lc), :],
                out_sems.at[slot])
            cp_out.start()
            local.append(cp_out)

        for j in range(PRE):
            local_chunk(j)

        fwd_base = (1 - my_x) * m_per + my_y * half
        hop2 = []
        nxt = PRE
        for k in range(NC):
            hop1[k].wait_recv()
            f = pltpu.make_async_remote_copy(
                src_ref=out_ref.at[pl.ds(fwd_base + k * hc, hc), :],
                dst_ref=out_ref.at[pl.ds(fwd_base + k * hc, hc), :],
                send_sem=send2_sems.at[k],
                recv_sem=recv2_sems.at[k],
                device_id=(my_x, 1 - my_y),
                device_id_type=pl.DeviceIdType.MESH,
            )
            f.start()
            hop2.append(f)
            take = 2 if nxt + 2 <= LC else LC - nxt
            for _ in range(take):
                local_chunk(nxt)
                nxt += 1
        while nxt < LC:
            local_chunk(nxt)
            nxt += 1

        for k in range(NC):
            hop1[k].wait_send()
        for k in range(NC):
            hop2[k].wait()
        for j in range(LC - NSLOT, LC):
            local[j].wait()

    return pl.pallas_call(
        body,
        out_shape=jax.ShapeDtypeStruct((2 * m_per, n), x.dtype),
        in_specs=[pl.BlockSpec(memory_space=pl.ANY)],
        out_specs=pl.BlockSpec(memory_space=pl.ANY),
        scratch_shapes=[
            pltpu.VMEM((NSLOT, m_per // LC, n), x.dtype),
            pltpu.SemaphoreType.DMA((NSLOT,)),
            pltpu.SemaphoreType.DMA((NSLOT,)),
            pltpu.SemaphoreType.DMA((NC,)),
            pltpu.SemaphoreType.DMA((NC,)),
            pltpu.SemaphoreType.DMA((NC,)),
            pltpu.SemaphoreType.DMA((NC,)),
        ],
        compiler_params=pltpu.CompilerParams(collective_id=0),
    )(x)


# device time: 465388 ns/iter; 4.5714x vs baseline; 1.0229x over previous
import jax
import jax.numpy as jnp
from jax import lax
from jax.experimental import pallas as pl
from jax.experimental.pallas import tpu as pltpu

NC = 32
LC = 32
NSLOT = 4
PRE = 4


def kernel(x):
    m_per, n = x.shape
    half = m_per // 2
    hc = half // NC
    lc = m_per // LC

    def body(x_ref, out_ref, vmem, in_sems, out_sems,
             send1_sems, recv1_sems, send2_sems, recv2_sems):
        my_x = lax.axis_index("x")
        my_y = lax.axis_index("y")

        barrier_sem = pltpu.get_barrier_semaphore()
        pl.semaphore_signal(barrier_sem, inc=1,
                            device_id=(1 - my_x, my_y),
                            device_id_type=pl.DeviceIdType.MESH)
        pl.semaphore_signal(barrier_sem, inc=1,
                            device_id=(my_x, 1 - my_y),
                            device_id_type=pl.DeviceIdType.MESH)
        pl.semaphore_wait(barrier_sem, 2)

        send_base = my_y * half
        own_base = my_x * m_per + my_y * half
        hop1 = []
        for k in range(NC):
            r = pltpu.make_async_remote_copy(
                src_ref=x_ref.at[pl.ds(send_base + k * hc, hc), :],
                dst_ref=out_ref.at[pl.ds(own_base + k * hc, hc), :],
                send_sem=send1_sems.at[k],
                recv_sem=recv1_sems.at[k],
                device_id=(1 - my_x, my_y),
                device_id_type=pl.DeviceIdType.MESH,
            )
            r.start()
            hop1.append(r)

        local = []

        def local_chunk(j):
            slot = j % NSLOT
            if j >= NSLOT:
                local[j - NSLOT].wait()
            cp_in = pltpu.make_async_copy(
                x_ref.at[pl.ds(j * lc, lc), :],
                vmem.at[slot], in_sems.at[slot])
            cp_in.start()
            cp_in.wait()
            cp_out = pltpu.make_async_copy(
                vmem.at[slot],
                out_ref.at[pl.ds(my_x * m_per + j * lc, lc), :],
                out_sems.at[slot])
            cp_out.start()
            local.append(cp_out)

        for j in range(PRE):
            local_chunk(j)

        fwd_base = (1 - my_x) * m_per + my_y * half
        hop2 = []
        nxt = PRE
        for k in range(NC):
            hop1[k].wait_recv()
            f = pltpu.make_async_remote_copy(
                src_ref=out_ref.at[pl.ds(fwd_base + k * hc, hc), :],
                dst_ref=out_ref.at[pl.ds(fwd_base + k * hc, hc), :],
                send_sem=send2_sems.at[k],
                recv_sem=recv2_sems.at[k],
                device_id=(my_x, 1 - my_y),
                device_id_type=pl.DeviceIdType.MESH,
            )
            f.start()
            hop2.append(f)
            take = 1 if nxt + 1 <= LC else LC - nxt
            for _ in range(take):
                local_chunk(nxt)
                nxt += 1
        while nxt < LC:
            local_chunk(nxt)
            nxt += 1

        for k in range(NC):
            hop1[k].wait_send()
        for k in range(NC):
            hop2[k].wait()
        for j in range(LC - NSLOT, LC):
            local[j].wait()

    return pl.pallas_call(
        body,
        out_shape=jax.ShapeDtypeStruct((2 * m_per, n), x.dtype),
        in_specs=[pl.BlockSpec(memory_space=pl.ANY)],
        out_specs=pl.BlockSpec(memory_space=pltpu.MemorySpace.HBM),
        scratch_shapes=[
            pltpu.VMEM((NSLOT, m_per // LC, n), x.dtype),
            pltpu.SemaphoreType.DMA((NSLOT,)),
            pltpu.SemaphoreType.DMA((NSLOT,)),
            pltpu.SemaphoreType.DMA((NC,)),
            pltpu.SemaphoreType.DMA((NC,)),
            pltpu.SemaphoreType.DMA((NC,)),
            pltpu.SemaphoreType.DMA((NC,)),
        ],
        compiler_params=pltpu.CompilerParams(collective_id=0),
    )(x)


# device time: 465280 ns/iter; 4.5725x vs baseline; 1.0002x over previous
import jax
import jax.numpy as jnp
from jax import lax
from jax.experimental import pallas as pl
from jax.experimental.pallas import tpu as pltpu

SIZES = [64] * 4 + [256] * 30 + [128, 64, 32, 32]
NC = len(SIZES)
LC = 32
NSLOT = 4


def kernel(x):
    m_per, n = x.shape
    half = m_per // 2
    offs = [sum(SIZES[:k]) for k in range(NC)]
    lc = m_per // LC

    def body(x_ref, out_ref, vmem, in_sems, out_sems,
             send1_sems, recv1_sems, send2_sems, recv2_sems):
        my_x = lax.axis_index("x")
        my_y = lax.axis_index("y")

        barrier_sem = pltpu.get_barrier_semaphore()
        pl.semaphore_signal(barrier_sem, inc=1,
                            device_id=(1 - my_x, my_y),
                            device_id_type=pl.DeviceIdType.MESH)
        pl.semaphore_signal(barrier_sem, inc=1,
                            device_id=(my_x, 1 - my_y),
                            device_id_type=pl.DeviceIdType.MESH)
        pl.semaphore_wait(barrier_sem, 2)

        send_base = my_y * half
        own_base = my_x * m_per + my_y * half
        hop1 = []
        for k in range(NC):
            r = pltpu.make_async_remote_copy(
                src_ref=x_ref.at[pl.ds(send_base + offs[k], SIZES[k]), :],
                dst_ref=out_ref.at[pl.ds(own_base + offs[k], SIZES[k]), :],
                send_sem=send1_sems.at[k],
                recv_sem=recv1_sems.at[k],
                device_id=(1 - my_x, my_y),
                device_id_type=pl.DeviceIdType.MESH,
            )
            r.start()
            hop1.append(r)

        local = []

        def local_chunk(j):
            slot = j % NSLOT
            if j >= NSLOT:
                local[j - NSLOT].wait()
            cp_in = pltpu.make_async_copy(
                x_ref.at[pl.ds(j * lc, lc), :],
                vmem.at[slot], in_sems.at[slot])
            cp_in.start()
            cp_in.wait()
            cp_out = pltpu.make_async_copy(
                vmem.at[slot],
                out_ref.at[pl.ds(my_x * m_per + j * lc, lc), :],
                out_sems.at[slot])
            cp_out.start()
            local.append(cp_out)

        fwd_base = (1 - my_x) * m_per + my_y * half
        hop2 = []
        nxt = 0
        for k in range(NC):
            hop1[k].wait_recv()
            f = pltpu.make_async_remote_copy(
                src_ref=out_ref.at[pl.ds(fwd_base + offs[k], SIZES[k]), :],
                dst_ref=out_ref.at[pl.ds(fwd_base + offs[k], SIZES[k]), :],
                send_sem=send2_sems.at[k],
                recv_sem=recv2_sems.at[k],
                device_id=(my_x, 1 - my_y),
                device_id_type=pl.DeviceIdType.MESH,
            )
            f.start()
            hop2.append(f)
            if k >= 4:
                take = min(2, LC - nxt)
                for _ in range(take):
                    local_chunk(nxt)
                    nxt += 1
        while nxt < LC:
            local_chunk(nxt)
            nxt += 1

        for k in range(NC):
            hop1[k].wait_send()
        for k in range(NC):
            hop2[k].wait()
        for j in range(LC - NSLOT, LC):
            local[j].wait()

    return pl.pallas_call(
        body,
        out_shape=jax.ShapeDtypeStruct((2 * m_per, n), x.dtype),
        in_specs=[pl.BlockSpec(memory_space=pl.ANY)],
        out_specs=pl.BlockSpec(memory_space=pltpu.MemorySpace.HBM),
        scratch_shapes=[
            pltpu.VMEM((NSLOT, m_per // LC, n), x.dtype),
            pltpu.SemaphoreType.DMA((NSLOT,)),
            pltpu.SemaphoreType.DMA((NSLOT,)),
            pltpu.SemaphoreType.DMA((NC,)),
            pltpu.SemaphoreType.DMA((NC,)),
            pltpu.SemaphoreType.DMA((NC,)),
            pltpu.SemaphoreType.DMA((NC,)),
        ],
        compiler_params=pltpu.CompilerParams(collective_id=0),
    )(x)
